# baseline (speedup 1.0000x reference)
import jax
import jax.numpy as jnp
from jax.experimental import pallas as pl
from jax.experimental.pallas import tpu as pltpu

_B, _NZ, _NM, _FIN, _FQK = 8, 256, 256, 64, 32
_TI = 32


def _body(z_ref, m_ref, w_ref, out_ref):
    z_t = z_ref[0]
    m_b = m_ref[0]
    W = w_ref[...]
    scale = _FQK ** -0.5
    q = jax.lax.dot_general(z_t, W[:_FQK, :], (((1,), (1,)), ((), ())),
                            preferred_element_type=jnp.float32) * scale
    k = jax.lax.dot_general(m_b, W[_FQK:, :], (((1,), (1,)), ((), ())),
                            preferred_element_type=jnp.float32)
    aw = jax.lax.dot_general(q, k, (((1,), (1,)), ((), ())),
                             preferred_element_type=jnp.float32)
    ex = jnp.exp(aw)
    w = ex / jnp.sum(ex, axis=1, keepdims=True)

    diff = z_t[:, None, :] - m_b[None, :, :]
    out_ref[...] = (diff * diff * w[:, :, None]).reshape(_TI * _NM, _FIN)


def kernel(z, m, W):
    return pl.pallas_call(
        _body,
        grid=(_B, _NZ // _TI),
        in_specs=[
            pl.BlockSpec((1, _TI, _FIN), lambda b, t: (b, t, 0)),
            pl.BlockSpec((1, _NM, _FIN), lambda b, t: (b, 0, 0)),
            pl.BlockSpec((2 * _FQK, _FIN), lambda b, t: (0, 0)),
        ],
        out_specs=pl.BlockSpec((_TI * _NM, _FIN),
                               lambda b, t: (b * (_NZ // _TI) + t, 0)),
        out_shape=jax.ShapeDtypeStruct((_B * _NZ * _NM, _FIN), jnp.float32),
    )(z, m, W)


# speedup vs baseline: 6.8476x; 6.8476x over previous
import jax
import jax.numpy as jnp
from jax.experimental import pallas as pl
from jax.experimental.pallas import tpu as pltpu

_B, _NZ, _NM, _FIN, _FQK = 8, 256, 256, 64, 32
_TI = 128


def _body(zt_ref, mt_ref, w_ref, out_ref):
    zt = zt_ref[0]
    mt = mt_ref[0]
    W = w_ref[...]
    scale = _FQK ** -0.5
    qT = jax.lax.dot_general(W[:_FQK, :], zt, (((1,), (0,)), ((), ())),
                             preferred_element_type=jnp.float32) * scale
    kT = jax.lax.dot_general(W[_FQK:, :], mt, (((1,), (0,)), ((), ())),
                             preferred_element_type=jnp.float32)
    aw = jax.lax.dot_general(qT, kT, (((0,), (0,)), ((), ())),
                             preferred_element_type=jnp.float32)
    ex = jnp.exp(aw)
    w = ex / jnp.sum(ex, axis=1, keepdims=True)

    for i in range(_TI):
        wi = w[i:i + 1, :]
        zi = zt[:, i:i + 1]
        diff = zi - mt
        out_ref[:, i * _NM:(i + 1) * _NM] = diff * diff * wi


def kernel(z, m, W):
    zt = jnp.swapaxes(z, 1, 2)
    mt = jnp.swapaxes(m, 1, 2)
    outT = pl.pallas_call(
        _body,
        grid=(_B, _NZ // _TI),
        in_specs=[
            pl.BlockSpec((1, _FIN, _TI), lambda b, t: (b, 0, t)),
            pl.BlockSpec((1, _FIN, _NM), lambda b, t: (b, 0, 0)),
            pl.BlockSpec((2 * _FQK, _FIN), lambda b, t: (0, 0)),
        ],
        out_specs=pl.BlockSpec((_FIN, _TI * _NM),
                               lambda b, t: (0, b * (_NZ // _TI) + t)),
        out_shape=jax.ShapeDtypeStruct((_FIN, _B * _NZ * _NM), jnp.float32),
    )(zt, mt, W)
    return outT.T
